# Initial kernel scaffold; baseline (speedup 1.0000x reference)
import jax
import jax.numpy as jnp
from jax import lax
from jax.experimental import pallas as pl
from jax.experimental.pallas import tpu as pltpu
from jax.experimental.pallas import tpu_sc as plsc

N = 10000
E = 320000
F = 128
R = 64
CHUNK = 128
NSUB = 16
NCHUNKS = E // CHUNK
LO = NCHUNKS // NSUB
REM = NCHUNKS % NSUB
ROWS_PT = N // NSUB
DH = 16

_mesh = plsc.VectorSubcoreMesh(core_axis_name="c", subcore_axis_name="s")


def _tile_ranges(s):
    base = s * LO + jnp.maximum(s - (NSUB - REM), 0)
    nch = jnp.where(s >= NSUB - REM, LO + 1, LO)
    return base, nch


def _deg_body(src2d, dst2d, zhist, onesb, degs, degd, hist, idx, onesv, sem):
    del sem
    c = lax.axis_index("c")
    s = lax.axis_index("s")
    base, nch = _tile_ranges(s)
    row0 = s * ROWS_PT

    pltpu.sync_copy(zhist, hist.at[pl.ds(row0, ROWS_PT)])
    pltpu.sync_copy(onesb, onesv)
    plsc.subcore_barrier()

    def run(idx2d, out):
        pltpu.sync_copy(idx2d.at[pl.ds(base, LO + 1)], idx)

        def body(k, carry):
            pltpu.sync_copy(onesv, hist.at[idx.at[k]], add=True)
            return carry

        lax.fori_loop(0, nch, body, 0)
        plsc.subcore_barrier()
        pltpu.sync_copy(hist.at[pl.ds(row0, ROWS_PT)],
                        out.at[pl.ds(row0, ROWS_PT)])

    pl.when(c == 0)(lambda: run(src2d, degs))
    pl.when(c == 1)(lambda: run(dst2d, degd))


_deg_call = pl.kernel(
    _deg_body,
    out_type=[jax.ShapeDtypeStruct((N, DH), jnp.float32),
              jax.ShapeDtypeStruct((N, DH), jnp.float32)],
    mesh=_mesh,
    scratch_types=[
        pltpu.VMEM_SHARED((N, DH), jnp.float32),
        pltpu.VMEM((LO + 1, CHUNK), jnp.int32),
        pltpu.VMEM((CHUNK, DH), jnp.float32),
        pltpu.SemaphoreType.DMA,
    ],
)


def _dense_body(feat_ref, deg_ref, w1_ref, w2a_ref, w2b_ref, t0_ref, t1_ref):
    x = feat_ref[...] * lax.rsqrt(jnp.maximum(deg_ref[...], 1.0))
    sfull = jnp.dot(x, w1_ref[...], preferred_element_type=jnp.float32)
    z = jnp.dot(x, w2a_ref[...], preferred_element_type=jnp.float32) + w2b_ref[...]
    t = jnp.tanh(z)
    lp = jnp.log(jnp.maximum(jnp.abs(t), 1e-30))
    sg = (t < 0).astype(jnp.float32)
    t0_ref[...] = jnp.concatenate([sfull[:, :R], lp], axis=1)
    t1_ref[...] = jnp.concatenate([sfull[:, R:], sg], axis=1)


_BLK = 1000

_dense_call = pl.pallas_call(
    _dense_body,
    grid=(N // _BLK,),
    in_specs=[
        pl.BlockSpec((_BLK, F), lambda i: (i, 0)),
        pl.BlockSpec((_BLK, 1), lambda i: (i, 0)),
        pl.BlockSpec((F, F), lambda i: (0, 0)),
        pl.BlockSpec((F, R), lambda i: (0, 0)),
        pl.BlockSpec((1, R), lambda i: (0, 0)),
    ],
    out_specs=[
        pl.BlockSpec((_BLK, F), lambda i: (i, 0)),
        pl.BlockSpec((_BLK, F), lambda i: (i, 0)),
    ],
    out_shape=[jax.ShapeDtypeStruct((N, F), jnp.float32),
               jax.ShapeDtypeStruct((N, F), jnp.float32)],
)


def _agg_body(t0, t1, src2d, dst2d, zrows, acc0, acc1,
              acc, sidx, didx, rows, sem):
    c = lax.axis_index("c")
    s = lax.axis_index("s")
    base, nch = _tile_ranges(s)
    row0 = s * ROWS_PT

    pltpu.sync_copy(zrows, acc.at[pl.ds(row0, ROWS_PT)])
    plsc.subcore_barrier()

    def run(tbl, out):
        pltpu.sync_copy(src2d.at[pl.ds(base, LO + 1)], sidx)
        pltpu.sync_copy(dst2d.at[pl.ds(base, LO + 1)], didx)

        def body(k, carry):
            pltpu.async_copy(tbl.at[sidx.at[k]], rows, sem).wait()
            pltpu.sync_copy(rows, acc.at[didx.at[k]], add=True)
            return carry

        lax.fori_loop(0, nch, body, 0)
        plsc.subcore_barrier()
        pltpu.sync_copy(acc.at[pl.ds(row0, ROWS_PT)],
                        out.at[pl.ds(row0, ROWS_PT)])

    pl.when(c == 0)(lambda: run(t0, acc0))
    pl.when(c == 1)(lambda: run(t1, acc1))


_agg_call = pl.kernel(
    _agg_body,
    out_type=[jax.ShapeDtypeStruct((N, F), jnp.float32),
              jax.ShapeDtypeStruct((N, F), jnp.float32)],
    mesh=_mesh,
    scratch_types=[
        pltpu.VMEM_SHARED((N, F), jnp.float32),
        pltpu.VMEM((LO + 1, CHUNK), jnp.int32),
        pltpu.VMEM((LO + 1, CHUNK), jnp.int32),
        pltpu.VMEM((CHUNK, F), jnp.float32),
        pltpu.SemaphoreType.DMA,
    ],
)


def _final_body(a0_ref, a1_ref, deg_ref, v_ref, out_ref):
    a0 = a0_ref[...]
    a1 = a1_ref[...]
    indeg = deg_ref[...]
    h_sum = jnp.concatenate([a0[:, :R], a1[:, :R]], axis=1)
    lp = a0[:, R:]
    cnt = a1[:, R:]
    sign = 1.0 - 2.0 * (cnt - 2.0 * jnp.floor(cnt * 0.5))
    h_prod = sign * jnp.exp(lp) * (indeg > 0).astype(jnp.float32)
    r = h_sum + jnp.dot(h_prod, v_ref[...], preferred_element_type=jnp.float32)
    out_ref[...] = r * lax.rsqrt(jnp.maximum(indeg, 1.0))


_final_call = pl.pallas_call(
    _final_body,
    grid=(N // _BLK,),
    in_specs=[
        pl.BlockSpec((_BLK, F), lambda i: (i, 0)),
        pl.BlockSpec((_BLK, F), lambda i: (i, 0)),
        pl.BlockSpec((_BLK, 1), lambda i: (i, 0)),
        pl.BlockSpec((R, F), lambda i: (0, 0)),
    ],
    out_specs=pl.BlockSpec((_BLK, F), lambda i: (i, 0)),
    out_shape=jax.ShapeDtypeStruct((N, F), jnp.float32),
)


def kernel(feat, edge_index, w1, w2, v):
    src2d = edge_index[0].reshape(NCHUNKS, CHUNK)
    dst2d = edge_index[1].reshape(NCHUNKS, CHUNK)
    zhist = jnp.zeros((ROWS_PT, DH), jnp.float32)
    onesb = jnp.concatenate(
        [jnp.ones((CHUNK, 1), jnp.float32),
         jnp.zeros((CHUNK, DH - 1), jnp.float32)], axis=1)
    zrows = jnp.zeros((ROWS_PT, F), jnp.float32)

    degs, degd = _deg_call(src2d, dst2d, zhist, onesb)
    outdeg = degs[:, 0:1]
    indeg = degd[:, 0:1]
    t0, t1 = _dense_call(feat, outdeg, w1, w2[:F], w2[F:F + 1])
    a0, a1 = _agg_call(t0, t1, src2d, dst2d, zrows)
    return _final_call(a0, a1, indeg, v)


# speedup vs baseline: 6.4729x; 6.4729x over previous
import jax
import jax.numpy as jnp
from jax import lax
from jax.experimental import pallas as pl
from jax.experimental.pallas import tpu as pltpu
from jax.experimental.pallas import tpu_sc as plsc

N = 10000
E = 320000
F = 128
R = 64
CHUNK = 128
NSUB = 16
NB = 160
EPAD = NSUB * NB * CHUNK - E
IB = 32
NIB = NB // IB
ROWS_PT = N // NSUB
NA = N + 8
DH = 16

_mesh = plsc.VectorSubcoreMesh(core_axis_name="c", subcore_axis_name="s")
_sc_params = pltpu.CompilerParams(use_tc_tiling_on_sc=False)


def _deg_body(srcd3, dstd3, zhist, onesb, degs, degd, hist, idxb, onesv, sem):
    del sem
    c = lax.axis_index("c")
    s = lax.axis_index("s")
    row0 = s * ROWS_PT

    pltpu.sync_copy(zhist, hist.at[pl.ds(row0, ROWS_PT)])
    pltpu.sync_copy(onesb, onesv)
    plsc.subcore_barrier()

    def run(idx3, out):
        def outer(b, carry):
            pltpu.sync_copy(idx3.at[s, pl.ds(b * IB, IB)], idxb)

            def body(k, carry2):
                pltpu.sync_copy(onesv, hist.at[idxb.at[k]], add=True)
                return carry2

            return lax.fori_loop(0, IB, body, carry)

        lax.fori_loop(0, NIB, outer, 0)
        plsc.subcore_barrier()
        pltpu.sync_copy(hist.at[pl.ds(row0, ROWS_PT)],
                        out.at[pl.ds(row0, ROWS_PT)])

    pl.when(c == 0)(lambda: run(srcd3, degs))
    pl.when(c == 1)(lambda: run(dstd3, degd))


_deg_call = pl.kernel(
    _deg_body,
    out_type=[jax.ShapeDtypeStruct((N, DH), jnp.float32),
              jax.ShapeDtypeStruct((N, DH), jnp.float32)],
    mesh=_mesh,
    scratch_types=[
        pltpu.VMEM_SHARED((NA, DH), jnp.float32),
        pltpu.VMEM((IB, CHUNK), jnp.int32),
        pltpu.VMEM((CHUNK, DH), jnp.float32),
        pltpu.SemaphoreType.DMA,
    ],
    compiler_params=_sc_params,
)


def _dense_body(feat_ref, deg_ref, w1_ref, w2a_ref, w2b_ref, t0_ref, t1_ref):
    x = feat_ref[...] * lax.rsqrt(jnp.maximum(deg_ref[...], 1.0))
    sfull = jnp.dot(x, w1_ref[...], preferred_element_type=jnp.float32)
    z = jnp.dot(x, w2a_ref[...], preferred_element_type=jnp.float32) + w2b_ref[...]
    t = jnp.tanh(z)
    lp = jnp.log(jnp.maximum(jnp.abs(t), 1e-30))
    sg = (t < 0).astype(jnp.float32)
    t0_ref[...] = jnp.concatenate([sfull[:, :R], lp], axis=1)
    t1_ref[...] = jnp.concatenate([sfull[:, R:], sg], axis=1)


_BLK = 1000

_dense_call = pl.pallas_call(
    _dense_body,
    grid=(N // _BLK,),
    in_specs=[
        pl.BlockSpec((_BLK, F), lambda i: (i, 0)),
        pl.BlockSpec((_BLK, 1), lambda i: (i, 0)),
        pl.BlockSpec((F, F), lambda i: (0, 0)),
        pl.BlockSpec((F, R), lambda i: (0, 0)),
        pl.BlockSpec((1, R), lambda i: (0, 0)),
    ],
    out_specs=[
        pl.BlockSpec((_BLK, F), lambda i: (i, 0)),
        pl.BlockSpec((_BLK, F), lambda i: (i, 0)),
    ],
    out_shape=[jax.ShapeDtypeStruct((N, F), jnp.float32),
               jax.ShapeDtypeStruct((N, F), jnp.float32)],
)


def _agg_body(t0, t1, srca3, dsta3, zrows, acc0, acc1,
              acc, sidxb, didxb, rows, sem):
    c = lax.axis_index("c")
    s = lax.axis_index("s")
    row0 = s * ROWS_PT

    pltpu.sync_copy(zrows, acc.at[pl.ds(row0, ROWS_PT)])
    plsc.subcore_barrier()

    def run(tbl, out):
        def outer(b, carry):
            pltpu.sync_copy(srca3.at[s, pl.ds(b * IB, IB)], sidxb)
            pltpu.sync_copy(dsta3.at[s, pl.ds(b * IB, IB)], didxb)

            def body(k, carry2):
                pltpu.async_copy(tbl.at[sidxb.at[k]], rows, sem).wait()
                pltpu.sync_copy(rows, acc.at[didxb.at[k]], add=True)
                return carry2

            return lax.fori_loop(0, IB, body, carry)

        lax.fori_loop(0, NIB, outer, 0)
        plsc.subcore_barrier()
        pltpu.sync_copy(acc.at[pl.ds(row0, ROWS_PT)],
                        out.at[pl.ds(row0, ROWS_PT)])

    pl.when(c == 0)(lambda: run(t0, acc0))
    pl.when(c == 1)(lambda: run(t1, acc1))


_agg_call = pl.kernel(
    _agg_body,
    out_type=[jax.ShapeDtypeStruct((N, F), jnp.float32),
              jax.ShapeDtypeStruct((N, F), jnp.float32)],
    mesh=_mesh,
    scratch_types=[
        pltpu.VMEM_SHARED((NA, F), jnp.float32),
        pltpu.VMEM((IB, CHUNK), jnp.int32),
        pltpu.VMEM((IB, CHUNK), jnp.int32),
        pltpu.VMEM((CHUNK, F), jnp.float32),
        pltpu.SemaphoreType.DMA,
    ],
    compiler_params=_sc_params,
)


def _final_body(a0_ref, a1_ref, deg_ref, v_ref, out_ref):
    a0 = a0_ref[...]
    a1 = a1_ref[...]
    indeg = deg_ref[...]
    h_sum = jnp.concatenate([a0[:, :R], a1[:, :R]], axis=1)
    lp = a0[:, R:]
    cnt = a1[:, R:]
    sign = 1.0 - 2.0 * (cnt - 2.0 * jnp.floor(cnt * 0.5))
    h_prod = sign * jnp.exp(lp) * (indeg > 0).astype(jnp.float32)
    r = h_sum + jnp.dot(h_prod, v_ref[...], preferred_element_type=jnp.float32)
    out_ref[...] = r * lax.rsqrt(jnp.maximum(indeg, 1.0))


_final_call = pl.pallas_call(
    _final_body,
    grid=(N // _BLK,),
    in_specs=[
        pl.BlockSpec((_BLK, F), lambda i: (i, 0)),
        pl.BlockSpec((_BLK, F), lambda i: (i, 0)),
        pl.BlockSpec((_BLK, 1), lambda i: (i, 0)),
        pl.BlockSpec((R, F), lambda i: (0, 0)),
    ],
    out_specs=pl.BlockSpec((_BLK, F), lambda i: (i, 0)),
    out_shape=jax.ShapeDtypeStruct((N, F), jnp.float32),
)


def kernel(feat, edge_index, w1, w2, v):
    src = edge_index[0]
    dst = edge_index[1]
    padn = jnp.full((EPAD,), N, jnp.int32)
    srcd3 = jnp.concatenate([src, padn]).reshape(NSUB, NB, CHUNK)
    dstd3 = jnp.concatenate([dst, padn]).reshape(NSUB, NB, CHUNK)
    srca3 = jnp.concatenate(
        [src, jnp.zeros((EPAD,), jnp.int32)]).reshape(NSUB, NB, CHUNK)
    zhist = jnp.zeros((ROWS_PT, DH), jnp.float32)
    onesb = jnp.concatenate(
        [jnp.ones((CHUNK, 1), jnp.float32),
         jnp.zeros((CHUNK, DH - 1), jnp.float32)], axis=1)
    zrows = jnp.zeros((ROWS_PT, F), jnp.float32)

    degs, degd = _deg_call(srcd3, dstd3, zhist, onesb)
    outdeg = degs[:, 0:1]
    indeg = degd[:, 0:1]
    t0, t1 = _dense_call(feat, outdeg, w1, w2[:F], w2[F:F + 1])
    a0, a1 = _agg_call(t0, t1, srca3, dstd3, zrows)
    return _final_call(a0, a1, indeg, v)


# speedup vs baseline: 7.5742x; 1.1701x over previous
import jax
import jax.numpy as jnp
from jax import lax
from jax.experimental import pallas as pl
from jax.experimental.pallas import tpu as pltpu
from jax.experimental.pallas import tpu_sc as plsc

N = 10000
E = 320000
F = 128
R = 64
CHUNK = 128
NSUB = 16
NB = 160
EPAD = NSUB * NB * CHUNK - E
IB = 32
NIB = NB // IB
ROWS_PT = N // NSUB
NA = N + 8
DH = 16

_mesh = plsc.VectorSubcoreMesh(core_axis_name="c", subcore_axis_name="s")
_sc_params = pltpu.CompilerParams(use_tc_tiling_on_sc=False)


def _deg_body(srcd3, dstd3, zhist, onesb, degs, degd, hist, idxb, onesv, sem):
    del sem
    c = lax.axis_index("c")
    s = lax.axis_index("s")
    row0 = s * ROWS_PT

    pltpu.sync_copy(zhist, hist.at[pl.ds(row0, ROWS_PT)])
    pltpu.sync_copy(onesb, onesv)
    plsc.subcore_barrier()

    def run(idx3, out):
        def outer(b, carry):
            pltpu.sync_copy(idx3.at[s, pl.ds(b * IB, IB)], idxb)

            def body(k, carry2):
                pltpu.sync_copy(onesv, hist.at[idxb.at[k]], add=True)
                return carry2

            return lax.fori_loop(0, IB, body, carry)

        lax.fori_loop(0, NIB, outer, 0)
        plsc.subcore_barrier()
        pltpu.sync_copy(hist.at[pl.ds(row0, ROWS_PT)],
                        out.at[pl.ds(row0, ROWS_PT)])

    pl.when(c == 0)(lambda: run(srcd3, degs))
    pl.when(c == 1)(lambda: run(dstd3, degd))


_deg_call = pl.kernel(
    _deg_body,
    out_type=[jax.ShapeDtypeStruct((N, DH), jnp.float32),
              jax.ShapeDtypeStruct((N, DH), jnp.float32)],
    mesh=_mesh,
    scratch_types=[
        pltpu.VMEM_SHARED((NA, DH), jnp.float32),
        pltpu.VMEM((IB, CHUNK), jnp.int32),
        pltpu.VMEM((CHUNK, DH), jnp.float32),
        pltpu.SemaphoreType.DMA,
    ],
    compiler_params=_sc_params,
)


def _dense_body(feat_ref, deg_ref, w1_ref, w2a_ref, w2b_ref, t0_ref, t1_ref):
    x = feat_ref[...] * lax.rsqrt(jnp.maximum(deg_ref[...], 1.0))
    sfull = jnp.dot(x, w1_ref[...], preferred_element_type=jnp.float32)
    z = jnp.dot(x, w2a_ref[...], preferred_element_type=jnp.float32) + w2b_ref[...]
    t = jnp.tanh(z)
    lp = jnp.log(jnp.maximum(jnp.abs(t), 1e-30))
    sg = (t < 0).astype(jnp.float32)
    t0_ref[...] = jnp.concatenate([sfull[:, :R], lp], axis=1)
    t1_ref[...] = jnp.concatenate([sfull[:, R:], sg], axis=1)


_BLK = 1000

_dense_call = pl.pallas_call(
    _dense_body,
    grid=(N // _BLK,),
    in_specs=[
        pl.BlockSpec((_BLK, F), lambda i: (i, 0)),
        pl.BlockSpec((_BLK, 1), lambda i: (i, 0)),
        pl.BlockSpec((F, F), lambda i: (0, 0)),
        pl.BlockSpec((F, R), lambda i: (0, 0)),
        pl.BlockSpec((1, R), lambda i: (0, 0)),
    ],
    out_specs=[
        pl.BlockSpec((_BLK, F), lambda i: (i, 0)),
        pl.BlockSpec((_BLK, F), lambda i: (i, 0)),
    ],
    out_shape=[jax.ShapeDtypeStruct((N, F), jnp.float32),
               jax.ShapeDtypeStruct((N, F), jnp.float32)],
)


def _agg_body(t0, t1, srca3, dsta3, zrows, acc0, acc1,
              acc, sidxb, didxb, rows0, rows1, sem0, sem1):
    c = lax.axis_index("c")
    s = lax.axis_index("s")
    row0 = s * ROWS_PT

    pltpu.sync_copy(zrows, acc.at[pl.ds(row0, ROWS_PT)])
    plsc.subcore_barrier()

    def run(tbl, out):
        def outer(b, carry):
            pltpu.sync_copy(srca3.at[s, pl.ds(b * IB, IB)], sidxb)
            pltpu.sync_copy(dsta3.at[s, pl.ds(b * IB, IB)], didxb)
            pltpu.async_copy(tbl.at[sidxb.at[0]], rows0, sem0)

            def body(i, carry2):
                k0 = 2 * i
                k1 = k0 + 1
                pltpu.async_copy(tbl.at[sidxb.at[k1]], rows1, sem1)
                pltpu.make_async_copy(tbl.at[sidxb.at[k0]], rows0, sem0).wait()
                pltpu.sync_copy(rows0, acc.at[didxb.at[k0]], add=True)
                def _prefetch():
                    pltpu.async_copy(tbl.at[sidxb.at[k0 + 2]], rows0, sem0)

                pl.when(i < IB // 2 - 1)(_prefetch)
                pltpu.make_async_copy(tbl.at[sidxb.at[k1]], rows1, sem1).wait()
                pltpu.sync_copy(rows1, acc.at[didxb.at[k1]], add=True)
                return carry2

            return lax.fori_loop(0, IB // 2, body, carry)

        lax.fori_loop(0, NIB, outer, 0)
        plsc.subcore_barrier()
        pltpu.sync_copy(acc.at[pl.ds(row0, ROWS_PT)],
                        out.at[pl.ds(row0, ROWS_PT)])

    pl.when(c == 0)(lambda: run(t0, acc0))
    pl.when(c == 1)(lambda: run(t1, acc1))


_agg_call = pl.kernel(
    _agg_body,
    out_type=[jax.ShapeDtypeStruct((N, F), jnp.float32),
              jax.ShapeDtypeStruct((N, F), jnp.float32)],
    mesh=_mesh,
    scratch_types=[
        pltpu.VMEM_SHARED((NA, F), jnp.float32),
        pltpu.VMEM((IB, CHUNK), jnp.int32),
        pltpu.VMEM((IB, CHUNK), jnp.int32),
        pltpu.VMEM((CHUNK, F), jnp.float32),
        pltpu.VMEM((CHUNK, F), jnp.float32),
        pltpu.SemaphoreType.DMA,
        pltpu.SemaphoreType.DMA,
    ],
    compiler_params=_sc_params,
)


def _final_body(a0_ref, a1_ref, deg_ref, v_ref, out_ref):
    a0 = a0_ref[...]
    a1 = a1_ref[...]
    indeg = deg_ref[...]
    h_sum = jnp.concatenate([a0[:, :R], a1[:, :R]], axis=1)
    lp = a0[:, R:]
    cnt = a1[:, R:]
    sign = 1.0 - 2.0 * (cnt - 2.0 * jnp.floor(cnt * 0.5))
    h_prod = sign * jnp.exp(lp) * (indeg > 0).astype(jnp.float32)
    r = h_sum + jnp.dot(h_prod, v_ref[...], preferred_element_type=jnp.float32)
    out_ref[...] = r * lax.rsqrt(jnp.maximum(indeg, 1.0))


_final_call = pl.pallas_call(
    _final_body,
    grid=(N // _BLK,),
    in_specs=[
        pl.BlockSpec((_BLK, F), lambda i: (i, 0)),
        pl.BlockSpec((_BLK, F), lambda i: (i, 0)),
        pl.BlockSpec((_BLK, 1), lambda i: (i, 0)),
        pl.BlockSpec((R, F), lambda i: (0, 0)),
    ],
    out_specs=pl.BlockSpec((_BLK, F), lambda i: (i, 0)),
    out_shape=jax.ShapeDtypeStruct((N, F), jnp.float32),
)


def kernel(feat, edge_index, w1, w2, v):
    src = edge_index[0]
    dst = edge_index[1]
    padn = jnp.full((EPAD,), N, jnp.int32)
    srcd3 = jnp.concatenate([src, padn]).reshape(NSUB, NB, CHUNK)
    dstd3 = jnp.concatenate([dst, padn]).reshape(NSUB, NB, CHUNK)
    srca3 = jnp.concatenate(
        [src, jnp.zeros((EPAD,), jnp.int32)]).reshape(NSUB, NB, CHUNK)
    zhist = jnp.zeros((ROWS_PT, DH), jnp.float32)
    onesb = jnp.concatenate(
        [jnp.ones((CHUNK, 1), jnp.float32),
         jnp.zeros((CHUNK, DH - 1), jnp.float32)], axis=1)
    zrows = jnp.zeros((ROWS_PT, F), jnp.float32)

    degs, degd = _deg_call(srcd3, dstd3, zhist, onesb)
    outdeg = degs[:, 0:1]
    indeg = degd[:, 0:1]
    t0, t1 = _dense_call(feat, outdeg, w1, w2[:F], w2[F:F + 1])
    a0, a1 = _agg_call(t0, t1, srca3, dstd3, zrows)
    return _final_call(a0, a1, indeg, v)
